# baseline (speedup 1.0000x reference)
import functools

import jax
import jax.numpy as jnp
from jax import lax
from jax.experimental import pallas as pl
from jax.experimental.pallas import tpu as pltpu
from jax.experimental.pallas import tpu_sc as plsc

MAX_LEN = 8192
HALF = MAX_LEN // 2
D_MODEL = 768
BATCH = 4
NUM_CORES = 2
NUM_SUBCORES = 16
NW = NUM_CORES * NUM_SUBCORES
ROWS_PER_W = HALF // NW
BCAST_ROWS = 256

_mesh = plsc.VectorSubcoreMesh(core_axis_name="c", subcore_axis_name="s")


@functools.partial(
    pl.kernel,
    mesh=_mesh,
    out_type=jax.ShapeDtypeStruct((MAX_LEN, D_MODEL), jnp.float32),
    scratch_types=[
        pltpu.VMEM((ROWS_PER_W, D_MODEL), jnp.float32),
        pltpu.VMEM((ROWS_PER_W,), jnp.int32),
        pltpu.VMEM((16, D_MODEL), jnp.float32),
        pltpu.VMEM((16,), jnp.int32),
        pltpu.VMEM((16,), jnp.int32),
        pltpu.SemaphoreType.DMA,
        pltpu.SemaphoreType.DMA,
    ],
)
def _rel_pos_gather(table_hbm, y_hbm, rows_v, ridx, spec_v, gidx, oidx,
                    sem, psem):
    wid = lax.axis_index("s") * NUM_CORES + lax.axis_index("c")
    rbase = wid * ROWS_PER_W
    pltpu.sync_copy(table_hbm.at[pl.ds(rbase, ROWS_PER_W)], rows_v)

    for t in range(ROWS_PER_W // 16):
        head = HALF - rbase - t * 16
        ridx[pl.ds(t * 16, 16)] = head - lax.iota(jnp.int32, 16)

    rev = pltpu.async_copy(rows_v, y_hbm.at[ridx], sem)
    fwd = pltpu.async_copy(
        rows_v, y_hbm.at[pl.ds(HALF + rbase, ROWS_PER_W)], sem)

    @pl.when(wid == 0)
    def _patch():
        gidx[...] = HALF - lax.iota(jnp.int32, 16)
        oidx[...] = lax.iota(jnp.int32, 16)
        pltpu.async_copy(table_hbm.at[gidx], spec_v, psem).wait()
        pltpu.async_copy(spec_v, y_hbm.at[oidx], psem).wait()

    rev.wait()
    fwd.wait()


def _bcast_body(y_ref, out_ref):
    out_ref[...] = jnp.broadcast_to(
        y_ref[...][None], (BATCH, BCAST_ROWS, D_MODEL))


_bcast = pl.pallas_call(
    _bcast_body,
    grid=(MAX_LEN // BCAST_ROWS,),
    in_specs=[pl.BlockSpec((BCAST_ROWS, D_MODEL), lambda i: (i, 0))],
    out_specs=pl.BlockSpec((BATCH, BCAST_ROWS, D_MODEL), lambda i: (0, i, 0)),
    out_shape=jax.ShapeDtypeStruct((BATCH, MAX_LEN, D_MODEL), jnp.float32),
)


def kernel(x, table):
    del x
    return _bcast(_rel_pos_gather(table))


# speedup vs baseline: 1.3856x; 1.3856x over previous
import functools

import jax
import jax.numpy as jnp
from jax import lax
from jax.experimental import pallas as pl
from jax.experimental.pallas import tpu as pltpu
from jax.experimental.pallas import tpu_sc as plsc

MAX_LEN = 8192
HALF = MAX_LEN // 2
D_MODEL = 768
BATCH = 4
NUM_CORES = 2
NUM_SUBCORES = 16
NW = NUM_CORES * NUM_SUBCORES
ROWS_PER_W = HALF // NW
CHUNK = ROWS_PER_W // 2

_mesh = plsc.VectorSubcoreMesh(core_axis_name="c", subcore_axis_name="s")


@functools.partial(
    pl.kernel,
    mesh=_mesh,
    out_type=jax.ShapeDtypeStruct((BATCH * MAX_LEN, D_MODEL), jnp.float32),
    scratch_types=[
        pltpu.VMEM((CHUNK, D_MODEL), jnp.float32),
        pltpu.VMEM((CHUNK, D_MODEL), jnp.float32),
        pltpu.VMEM((CHUNK,), jnp.int32),
        pltpu.VMEM((CHUNK,), jnp.int32),
        pltpu.VMEM((CHUNK,), jnp.int32),
        pltpu.VMEM((CHUNK,), jnp.int32),
        pltpu.VMEM((CHUNK,), jnp.int32),
        pltpu.VMEM((CHUNK,), jnp.int32),
        pltpu.VMEM((CHUNK,), jnp.int32),
        pltpu.VMEM((CHUNK,), jnp.int32),
        pltpu.VMEM((16, D_MODEL), jnp.float32),
        pltpu.VMEM((16,), jnp.int32),
        pltpu.VMEM((16,), jnp.int32),
        pltpu.SemaphoreType.DMA,
        pltpu.SemaphoreType.DMA,
        pltpu.SemaphoreType.DMA,
    ],
)
def _rel_pos_emb(table_hbm, out_hbm, rows_a, rows_b,
                 ia0, ia1, ia2, ia3, ib0, ib1, ib2, ib3,
                 spec_v, gidx, oidx, sem, psem, rsem):
    wid = lax.axis_index("s") * NUM_CORES + lax.axis_index("c")
    rbase = wid * ROWS_PER_W

    rows = [rows_a, rows_b]
    ridx = [[ia0, ia1, ia2, ia3], [ib0, ib1, ib2, ib3]]

    def read_desc(c):
        return pltpu.make_async_copy(
            table_hbm.at[pl.ds(rbase + c * CHUNK, CHUNK)], rows[c], rsem)

    read_desc(0).start()
    read_desc(1).start()

    for c in range(2):
        for b in range(BATCH):
            for t in range(CHUNK // 16):
                head = b * MAX_LEN + HALF - rbase - c * CHUNK - t * 16
                ridx[c][b][pl.ds(t * 16, 16)] = head - lax.iota(jnp.int32, 16)

    copies = []
    for c in range(2):
        read_desc(c).wait()
        for b in range(BATCH):
            copies.append(pltpu.async_copy(rows[c], out_hbm.at[ridx[c][b]],
                                           sem))
            copies.append(pltpu.async_copy(
                rows[c],
                out_hbm.at[pl.ds(b * MAX_LEN + HALF + rbase + c * CHUNK,
                                 CHUNK)],
                sem))

    @pl.when(wid < BATCH)
    def _patch():
        gidx[...] = HALF - lax.iota(jnp.int32, 16)
        oidx[...] = wid * MAX_LEN + lax.iota(jnp.int32, 16)
        pltpu.async_copy(table_hbm.at[gidx], spec_v, psem).wait()
        pltpu.async_copy(spec_v, out_hbm.at[oidx], psem).wait()

    for c in copies:
        c.wait()


def kernel(x, table):
    del x
    out = _rel_pos_emb(table)
    return out.reshape(BATCH, MAX_LEN, D_MODEL)
